# baseline (speedup 1.0000x reference)
import jax
import jax.numpy as jnp
from jax.experimental import pallas as pl
from jax.experimental.pallas import tpu as pltpu

LANE = 128
_FOLD = 32


def _linear_compact_kernel(x_ref, w_ref, b_ref, o_ref):
    acc = jnp.dot(x_ref[...], w_ref[...], preferred_element_type=jnp.float32)
    o_ref[...] = (acc + b_ref[...]).astype(o_ref.dtype)


def kernel(x, w, b):
    B, D_in = x.shape
    D_out = w.shape[0]
    fold = _FOLD
    assert D_out * fold == LANE and B % fold == 0

    x_r = x.reshape(B // fold, fold * D_in)

    w_t = w.T.astype(x.dtype)
    eye = jnp.eye(fold, dtype=x.dtype)
    w_big = (eye[:, None, :, None] * w_t[None, :, None, :]).reshape(
        fold * D_in, fold * D_out)
    b_big = jnp.tile(b.astype(x.dtype), fold)[None, :]

    rows = B // fold
    tb = 128
    n_tiles = rows // tb

    out = pl.pallas_call(
        _linear_compact_kernel,
        out_shape=jax.ShapeDtypeStruct((rows, fold * D_out), x.dtype),
        grid_spec=pltpu.PrefetchScalarGridSpec(
            num_scalar_prefetch=0,
            grid=(n_tiles,),
            in_specs=[
                pl.BlockSpec((tb, fold * D_in), lambda i: (i, 0)),
                pl.BlockSpec((fold * D_in, fold * D_out), lambda i: (0, 0)),
                pl.BlockSpec((1, fold * D_out), lambda i: (0, 0)),
            ],
            out_specs=pl.BlockSpec((tb, fold * D_out), lambda i: (i, 0)),
        ),
        compiler_params=pltpu.CompilerParams(
            dimension_semantics=("parallel",),
        ),
    )(x_r, w_big, b_big)

    return out.reshape(B, D_out)


# speedup vs baseline: 1.0135x; 1.0135x over previous
import jax
import jax.numpy as jnp
from jax.experimental import pallas as pl
from jax.experimental.pallas import tpu as pltpu

LANE = 128
_FOLD = 32


def _linear_compact_kernel(x_ref, w_ref, b_ref, o_ref):
    acc = jnp.dot(x_ref[...], w_ref[...], preferred_element_type=jnp.float32)
    o_ref[...] = (acc + b_ref[...]).astype(o_ref.dtype)


def kernel(x, w, b):
    B, D_in = x.shape
    D_out = w.shape[0]
    fold = _FOLD
    assert D_out * fold == LANE and B % fold == 0

    x_r = x.reshape(B // fold, fold * D_in)

    p_full = jnp.tile(w.T.astype(x.dtype), (1, fold))
    e_mask = (jnp.arange(fold, dtype=jnp.int32)[:, None]
              == (jnp.arange(fold * D_out, dtype=jnp.int32)[None, :] // D_out)
              ).astype(x.dtype)
    w_big = (e_mask[:, None, :] * p_full[None, :, :]).reshape(
        fold * D_in, fold * D_out)
    b_big = jnp.tile(b.astype(x.dtype), fold)[None, :]

    rows = B // fold
    tb = 128
    n_tiles = rows // tb

    out = pl.pallas_call(
        _linear_compact_kernel,
        out_shape=jax.ShapeDtypeStruct((rows, fold * D_out), x.dtype),
        grid_spec=pltpu.PrefetchScalarGridSpec(
            num_scalar_prefetch=0,
            grid=(n_tiles,),
            in_specs=[
                pl.BlockSpec((tb, fold * D_in), lambda i: (i, 0)),
                pl.BlockSpec((fold * D_in, fold * D_out), lambda i: (0, 0)),
                pl.BlockSpec((1, fold * D_out), lambda i: (0, 0)),
            ],
            out_specs=pl.BlockSpec((tb, fold * D_out), lambda i: (i, 0)),
        ),
        compiler_params=pltpu.CompilerParams(
            dimension_semantics=("parallel",),
        ),
    )(x_r, w_big, b_big)

    return out.reshape(B, D_out)


# speedup vs baseline: 2.3383x; 2.3072x over previous
import jax
import jax.numpy as jnp
from jax.experimental import pallas as pl
from jax.experimental.pallas import tpu as pltpu

LANE = 128


def _linear_kernel(x_ref, w_ref, b_ref, o_ref):
    d_out = o_ref.shape[-1]
    acc = jnp.dot(x_ref[...], w_ref[...], preferred_element_type=jnp.float32)
    o_ref[...] = (acc + b_ref[...])[:, :d_out].astype(o_ref.dtype)


def kernel(x, w, b):
    B, D_in = x.shape
    D_out = w.shape[0]

    w_t = jnp.zeros((D_in, LANE), x.dtype).at[:, :D_out].set(w.T.astype(x.dtype))
    b_p = jnp.zeros((1, LANE), x.dtype).at[0, :D_out].set(b.astype(x.dtype))

    tb = 4096
    n_tiles = B // tb

    out = pl.pallas_call(
        _linear_kernel,
        out_shape=jax.ShapeDtypeStruct((B, D_out), x.dtype),
        grid_spec=pltpu.PrefetchScalarGridSpec(
            num_scalar_prefetch=0,
            grid=(n_tiles,),
            in_specs=[
                pl.BlockSpec((tb, D_in), lambda i: (i, 0)),
                pl.BlockSpec((D_in, LANE), lambda i: (0, 0)),
                pl.BlockSpec((1, LANE), lambda i: (0, 0)),
            ],
            out_specs=pl.BlockSpec((tb, D_out), lambda i: (i, 0)),
        ),
        compiler_params=pltpu.CompilerParams(
            dimension_semantics=("parallel",),
        ),
    )(x, w_t, b_p)
    return out
